# baseline (speedup 1.0000x reference)
import jax
import jax.numpy as jnp
from jax import lax
from jax.experimental import pallas as pl
from jax.experimental.pallas import tpu as pltpu
from jax.experimental.pallas import tpu_sc as plsc

B = 16384
L = 200
E = 32
V = 257

NC = 2
NS = 16
NW = NC * NS
LANES = 16

BPT = B // NW
BT = 128
LO = 8
NBT = BPT // BT
NLO = L // LO
ROUNDS = NBT * NLO
IPR = BT * LO
CHUNKS = IPR // LANES
SROW = 33


def _body(idx_hbm, table_hbm, out_hbm, table_v, idx_v0, idx_v1,
          stage, ov0, ov1, sem_in0, sem_in1, sem_out0, sem_out1):
    wid = lax.axis_index("s") * NC + lax.axis_index("c")
    b_tile0 = wid * BPT

    pltpu.sync_copy(table_hbm, table_v)

    def idx_dma_start(r, ib, sem):
        t = r // NLO
        lo = r % NLO
        b0 = b_tile0 + t * BT
        for k in range(LO):
            pltpu.async_copy(
                idx_hbm.at[pl.ds((lo * LO + k) * B + b0, BT)],
                ib.at[pl.ds(k * BT, BT)], sem)

    def idx_dma_wait(ib, sem):
        for k in range(LO):
            pltpu.make_async_copy(
                idx_hbm.at[pl.ds(0, BT)], ib.at[pl.ds(k * BT, BT)],
                sem).wait()

    idx_dma_start(0, idx_v0, sem_in0)
    idx_dma_start(1, idx_v1, sem_in1)

    def compute_block(ib, ov):
        @plsc.parallel_loop(0, CHUNKS, 1, unroll=8)
        def _(c):
            iv = ib[pl.ds(c * LANES, LANES)]
            base = c * (LANES * SROW)
            for j in range(LANES):
                s = iv[j] * E
                r0 = table_v[pl.ds(s, LANES)]
                r1 = table_v[pl.ds(s + LANES, LANES)]
                a = base + j * SROW
                stage[pl.ds(a, LANES)] = r0
                stage[pl.ds(a + LANES, LANES)] = r1

        @plsc.parallel_loop(0, CHUNKS, 1, unroll=8)
        def _(c2):
            lo_off = c2 // 8
            bq = (c2 % 8) * LANES
            lane = lax.iota(jnp.int32, LANES)
            base_v = (c2 * LANES + lane) * SROW
            bv = bq + lane
            for e in range(E):
                vals = plsc.load_gather(stage, [base_v + e])
                plsc.store_scatter(
                    ov, [jnp.full((LANES,), e, jnp.int32),
                         jnp.full((LANES,), 1, jnp.int32) * lo_off, bv],
                    vals)

    def pair_body(i, carry):
        for sl, ib, ov, sem_in, sem_out in (
                (0, idx_v0, ov0, sem_in0, sem_out0),
                (1, idx_v1, ov1, sem_in1, sem_out1)):
            g = 2 * i + sl
            idx_dma_wait(ib, sem_in)

            @pl.when(g >= 2)
            def _():
                pltpu.make_async_copy(
                    ov, out_hbm.at[:, pl.ds(0, LO), pl.ds(0, BT)],
                    sem_out).wait()

            compute_block(ib, ov)

            t = g // NLO
            lo = g % NLO
            b0 = b_tile0 + t * BT
            pltpu.async_copy(
                ov, out_hbm.at[:, pl.ds(lo * LO, LO), pl.ds(b0, BT)],
                sem_out)

            @pl.when(g + 2 < ROUNDS)
            def _():
                g2 = jnp.minimum(g + 2, ROUNDS - 1)
                idx_dma_start(g2, ib, sem_in)
        return carry

    lax.fori_loop(0, ROUNDS // 2, pair_body, 0)

    pltpu.make_async_copy(
        ov0, out_hbm.at[:, pl.ds(0, LO), pl.ds(0, BT)], sem_out0).wait()
    pltpu.make_async_copy(
        ov1, out_hbm.at[:, pl.ds(0, LO), pl.ds(0, BT)], sem_out1).wait()


def kernel(input_x, table):
    xt = input_x.reshape(B, L).astype(jnp.int32).T.reshape(-1)
    table_r = table.astype(jnp.float32).reshape(-1)

    mesh = plsc.VectorSubcoreMesh(
        core_axis_name="c", subcore_axis_name="s",
        num_cores=NC, num_subcores=NS,
    )
    out = pl.kernel(
        _body,
        out_type=jax.ShapeDtypeStruct((E, L, B), jnp.float32),
        mesh=mesh,
        compiler_params=pltpu.CompilerParams(needs_layout_passes=False),
        scratch_types=[
            pltpu.VMEM((V * E,), jnp.float32),
            pltpu.VMEM((IPR,), jnp.int32),
            pltpu.VMEM((IPR,), jnp.int32),
            pltpu.VMEM((IPR * SROW,), jnp.float32),
            pltpu.VMEM((E, LO, BT), jnp.float32),
            pltpu.VMEM((E, LO, BT), jnp.float32),
            pltpu.SemaphoreType.DMA,
            pltpu.SemaphoreType.DMA,
            pltpu.SemaphoreType.DMA,
            pltpu.SemaphoreType.DMA,
        ],
    )(xt, table_r)
    return jnp.transpose(out, (2, 0, 1))


# speedup vs baseline: 1.1513x; 1.1513x over previous
import jax
import jax.numpy as jnp
from jax import lax
from jax.experimental import pallas as pl
from jax.experimental.pallas import tpu as pltpu
from jax.experimental.pallas import tpu_sc as plsc

B = 16384
L = 200
E = 32
V = 257

NC = 2
NS = 16
NW = NC * NS
LANES = 16

BPT = B // NW
BT = 128
LO = 8
NBT = BPT // BT
NLO = L // LO
ROUNDS = NBT * NLO
IPR = BT * LO
CHUNKS = IPR // LANES
SROW = 33


def _body(idx_hbm, table_hbm, out_hbm, table_v, idx_v0, idx_v1,
          stage, ov0, ov1, sem_in0, sem_in1, sem_out0, sem_out1):
    wid = lax.axis_index("s") * NC + lax.axis_index("c")
    b_tile0 = wid * BPT

    pltpu.sync_copy(table_hbm, table_v)

    def idx_dma_start(r, ib, sem):
        t = r // NLO
        lo = r % NLO
        b0 = b_tile0 + t * BT
        for k in range(LO):
            pltpu.async_copy(
                idx_hbm.at[pl.ds((lo * LO + k) * B + b0, BT)],
                ib.at[pl.ds(k * BT, BT)], sem)

    def idx_dma_wait(ib, sem):
        for k in range(LO):
            pltpu.make_async_copy(
                idx_hbm.at[pl.ds(0, BT)], ib.at[pl.ds(k * BT, BT)],
                sem).wait()

    idx_dma_start(0, idx_v0, sem_in0)
    idx_dma_start(1, idx_v1, sem_in1)

    def compute_block(ib, ov):
        @plsc.parallel_loop(0, CHUNKS, 1, unroll=4)
        def _(c):
            iv = ib[pl.ds(c * LANES, LANES)]
            base = c * (LANES * SROW)
            for j in range(LANES):
                s = iv[j] * E
                r0 = table_v[pl.ds(s, LANES)]
                r1 = table_v[pl.ds(s + LANES, LANES)]
                a = base + j * SROW
                stage[pl.ds(a, LANES)] = r0
                stage[pl.ds(a + LANES, LANES)] = r1

        @plsc.parallel_loop(0, CHUNKS, 1, unroll=4)
        def _(c2):
            lo_off = c2 // 8
            bq = (c2 % 8) * LANES
            lane = lax.iota(jnp.int32, LANES)
            base_v = (c2 * LANES + lane) * SROW
            bv = bq + lane
            for e in range(E):
                vals = plsc.load_gather(stage, [base_v + e])
                plsc.store_scatter(
                    ov, [jnp.full((LANES,), e, jnp.int32),
                         jnp.full((LANES,), 1, jnp.int32) * lo_off, bv],
                    vals)

    def pair_body(i, carry):
        for sl, ib, ov, sem_in, sem_out in (
                (0, idx_v0, ov0, sem_in0, sem_out0),
                (1, idx_v1, ov1, sem_in1, sem_out1)):
            g = 2 * i + sl
            idx_dma_wait(ib, sem_in)

            @pl.when(g >= 2)
            def _():
                pltpu.make_async_copy(
                    ov, out_hbm.at[:, pl.ds(0, LO), pl.ds(0, BT)],
                    sem_out).wait()

            compute_block(ib, ov)

            t = g // NLO
            lo = g % NLO
            b0 = b_tile0 + t * BT
            pltpu.async_copy(
                ov, out_hbm.at[:, pl.ds(lo * LO, LO), pl.ds(b0, BT)],
                sem_out)

            @pl.when(g + 2 < ROUNDS)
            def _():
                g2 = jnp.minimum(g + 2, ROUNDS - 1)
                idx_dma_start(g2, ib, sem_in)
        return carry

    lax.fori_loop(0, ROUNDS // 2, pair_body, 0)

    pltpu.make_async_copy(
        ov0, out_hbm.at[:, pl.ds(0, LO), pl.ds(0, BT)], sem_out0).wait()
    pltpu.make_async_copy(
        ov1, out_hbm.at[:, pl.ds(0, LO), pl.ds(0, BT)], sem_out1).wait()


def kernel(input_x, table):
    xt = input_x.reshape(B, L).astype(jnp.int32).T.reshape(-1)
    table_r = table.astype(jnp.float32).reshape(-1)

    mesh = plsc.VectorSubcoreMesh(
        core_axis_name="c", subcore_axis_name="s",
        num_cores=NC, num_subcores=NS,
    )
    out = pl.kernel(
        _body,
        out_type=jax.ShapeDtypeStruct((E, L, B), jnp.float32),
        mesh=mesh,
        compiler_params=pltpu.CompilerParams(needs_layout_passes=False),
        scratch_types=[
            pltpu.VMEM((V * E,), jnp.float32),
            pltpu.VMEM((IPR,), jnp.int32),
            pltpu.VMEM((IPR,), jnp.int32),
            pltpu.VMEM((IPR * SROW,), jnp.float32),
            pltpu.VMEM((E, LO, BT), jnp.float32),
            pltpu.VMEM((E, LO, BT), jnp.float32),
            pltpu.SemaphoreType.DMA,
            pltpu.SemaphoreType.DMA,
            pltpu.SemaphoreType.DMA,
            pltpu.SemaphoreType.DMA,
        ],
    )(xt, table_r)
    return jnp.transpose(out, (2, 0, 1))
